# Initial kernel scaffold; baseline (speedup 1.0000x reference)
import jax
import jax.numpy as jnp
from jax.experimental import pallas as pl


def kernel(fpn0, fpn1, fpn2, fpn3, fpn4, cls_w, cls_b, cls_g, cls_beta, cls_fw, cls_fb, reg_w, reg_b, reg_g, reg_beta, reg_fw, reg_fb):
    raise NotImplementedError("write your pallas kernel here")


# speedup vs baseline: 3.1218x; 3.1218x over previous
import functools

import jax
import jax.numpy as jnp
from jax import lax
from jax.experimental import pallas as pl

IN_CH = 96
HID = 192
NUM_CLASSES = 80
OUT_ROWS = 88
NUM_GROUPS = 64
GN_EPS = 1e-05
STRIDES = (8, 16, 32, 64, 128)
SIZES = ((64, 64), (32, 32), (16, 16), (8, 8), (4, 4))


def _level_kernel(x_ref, w3_ref, g_ref, b3_ref, gam_ref, bet_ref,
                  wf_ref, fb_ref, cls_ref, cent_ref, reg_ref,
                  *, H, W, stride):
    S = H * W
    logw = W.bit_length() - 1
    x = x_ref[0].astype(jnp.bfloat16)
    pos = lax.broadcasted_iota(jnp.int32, (1, S), 1)
    col = pos & (W - 1)
    row = pos >> logw
    parts = []
    for dy in (-1, 0, 1):
        for dx in (-1, 0, 1):
            k = dy * W + dx
            sh = jnp.roll(x, -k, axis=1) if k else x
            valid = ((row + dy >= 0) & (row + dy < H)
                     & (col + dx >= 0) & (col + dx < W))
            parts.append(jnp.where(valid, sh, jnp.bfloat16(0)))
    xcol = jnp.concatenate(parts, axis=0)
    h = lax.dot_general(w3_ref[...], xcol, (((1,), (0,)), ((), ())),
                        preferred_element_type=jnp.float32)
    h = h + b3_ref[...]
    s1 = jnp.sum(h, axis=1, keepdims=True)
    s2 = jnp.sum(h * h, axis=1, keepdims=True)
    st = jnp.concatenate([s1, s2], axis=1)
    gs = lax.dot_general(g_ref[...], st, (((1,), (0,)), ((), ())),
                         preferred_element_type=jnp.float32,
                         precision=lax.Precision.HIGHEST)
    cnt = 1.0 / (3.0 * S)
    mean = gs[:, 0:1] * cnt
    var = gs[:, 1:2] * cnt - mean * mean
    inv = lax.rsqrt(var + GN_EPS)
    scale = inv * gam_ref[...]
    shift = bet_ref[...] - mean * scale
    hn = h * scale + shift
    a = hn * jax.nn.sigmoid(hn)
    y = lax.dot_general(wf_ref[...], a.astype(jnp.bfloat16),
                        (((1,), (0,)), ((), ())),
                        preferred_element_type=jnp.float32)
    y = y + fb_ref[...]
    cls_ref[0] = y[0:NUM_CLASSES]
    cent_ref[0] = y[NUM_CLASSES:NUM_CLASSES + 1]
    reg_ref[0] = jnp.maximum(y[NUM_CLASSES + 1:NUM_CLASSES + 5]
                             * jnp.float32(stride), 0.0)


def _run_level(x, W3, G, b3, gam, bet, Wf, fb, H, W, stride):
    B = x.shape[0]
    S = H * W
    xr = x.reshape(B, IN_CH, S)
    f32 = jnp.float32
    out_shape = (
        jax.ShapeDtypeStruct((B, NUM_CLASSES, S), f32),
        jax.ShapeDtypeStruct((B, 1, S), f32),
        jax.ShapeDtypeStruct((B, 4, S), f32),
    )
    full = lambda shp: pl.BlockSpec(shp, lambda b: (0,) * len(shp))
    cls, cent, reg = pl.pallas_call(
        functools.partial(_level_kernel, H=H, W=W, stride=stride),
        grid=(B,),
        in_specs=[
            pl.BlockSpec((1, IN_CH, S), lambda b: (b, 0, 0)),
            full((HID, 9 * IN_CH)),
            full((HID, HID)),
            full((HID, 1)),
            full((HID, 1)),
            full((HID, 1)),
            full((OUT_ROWS, HID)),
            full((OUT_ROWS, 1)),
        ],
        out_specs=(
            pl.BlockSpec((1, NUM_CLASSES, S), lambda b: (b, 0, 0)),
            pl.BlockSpec((1, 1, S), lambda b: (b, 0, 0)),
            pl.BlockSpec((1, 4, S), lambda b: (b, 0, 0)),
        ),
        out_shape=out_shape,
    )(xr, W3, G, b3, gam, bet, Wf, fb)
    return (cls.reshape(B, NUM_CLASSES, H, W),
            reg.reshape(B, 4, H, W),
            cent.reshape(B, 1, H, W))


def kernel(fpn0, fpn1, fpn2, fpn3, fpn4,
           cls_w, cls_b, cls_g, cls_beta, cls_fw, cls_fb,
           reg_w, reg_b, reg_g, reg_beta, reg_fw, reg_fb):
    f32 = jnp.float32
    wtap = jnp.concatenate([cls_w, reg_w], axis=0)
    W3 = jnp.transpose(wtap, (0, 2, 3, 1)).reshape(HID, 9 * IN_CH)
    W3 = W3.astype(jnp.bfloat16)
    b3 = jnp.concatenate([cls_b, reg_b]).reshape(HID, 1).astype(f32)
    gam = jnp.concatenate([cls_g, reg_g]).reshape(HID, 1).astype(f32)
    bet = jnp.concatenate([cls_beta, reg_beta]).reshape(HID, 1).astype(f32)
    gidx = jnp.arange(HID) // 3
    G = (gidx[:, None] == gidx[None, :]).astype(f32)
    Wf = jnp.zeros((OUT_ROWS, HID), f32)
    Wf = Wf.at[0:NUM_CLASSES, 0:IN_CH].set(cls_fw[:, :, 0, 0])
    Wf = Wf.at[NUM_CLASSES:NUM_CLASSES + 5, IN_CH:HID].set(reg_fw[:, :, 0, 0])
    Wf = Wf.astype(jnp.bfloat16)
    fb = jnp.zeros((OUT_ROWS, 1), f32)
    fb = fb.at[0:NUM_CLASSES, 0].set(cls_fb)
    fb = fb.at[NUM_CLASSES:NUM_CLASSES + 5, 0].set(reg_fb)

    cls_out, reg_out, cent_out = [], [], []
    for x, (H, W), stride in zip((fpn0, fpn1, fpn2, fpn3, fpn4),
                                 SIZES, STRIDES):
        c, r, ct = _run_level(x, W3, G, b3, gam, bet, Wf, fb, H, W, stride)
        cls_out.append(c)
        reg_out.append(r)
        cent_out.append(ct)
    return tuple(cls_out) + tuple(reg_out) + tuple(cent_out)


# speedup vs baseline: 3.5865x; 1.1489x over previous
import jax
import jax.numpy as jnp
from jax import lax
from jax.experimental import pallas as pl

IN_CH = 96
HID = 192
NUM_CLASSES = 80
OUT_ROWS = 88
GN_EPS = 1e-05
STRIDES = (8, 16, 32, 64, 128)
SIZES = ((64, 64), (32, 32), (16, 16), (8, 8), (4, 4))
NLEV = 5


def _do_level(x_ref, w3_ref, g2_ref, b3_ref, gam_ref, bet_ref,
              wf_ref, fb_ref, cls_ref, cent_ref, reg_ref, H, W, stride):
    S = H * W
    logw = W.bit_length() - 1
    x = x_ref[0].astype(jnp.bfloat16)
    pos = lax.broadcasted_iota(jnp.int32, (1, S), 1)
    col = pos & (W - 1)
    row = pos >> logw
    parts = []
    for dy in (-1, 0, 1):
        for dx in (-1, 0, 1):
            k = dy * W + dx
            sh = jnp.roll(x, -k, axis=1) if k else x
            valid = ((row + dy >= 0) & (row + dy < H)
                     & (col + dx >= 0) & (col + dx < W))
            parts.append(jnp.where(valid, sh, jnp.bfloat16(0)))
    xcol = jnp.concatenate(parts, axis=0)
    h = lax.dot_general(w3_ref[...], xcol, (((1,), (0,)), ((), ())),
                        preferred_element_type=jnp.float32)
    h = h + b3_ref[...]
    s1 = jnp.sum(h, axis=1, keepdims=True)
    s2 = jnp.sum(h * h, axis=1, keepdims=True)
    st = jnp.concatenate([s1, s2], axis=0)
    gs = lax.dot_general(g2_ref[...], st, (((1,), (0,)), ((), ())),
                         preferred_element_type=jnp.float32,
                         precision=lax.Precision.HIGHEST)
    cnt = 1.0 / (3.0 * S)
    mean = gs[0:HID] * cnt
    var = gs[HID:2 * HID] * cnt - mean * mean
    inv = lax.rsqrt(var + GN_EPS)
    scale = inv * gam_ref[...]
    shift = bet_ref[...] - mean * scale
    hn = h * scale + shift
    a = hn * jax.nn.sigmoid(hn)
    y = lax.dot_general(wf_ref[...], a.astype(jnp.bfloat16),
                        (((1,), (0,)), ((), ())),
                        preferred_element_type=jnp.float32)
    y = y + fb_ref[...]
    cls_ref[0] = y[0:NUM_CLASSES]
    cent_ref[0] = y[NUM_CLASSES:NUM_CLASSES + 1]
    reg_ref[0] = jnp.maximum(y[NUM_CLASSES + 1:NUM_CLASSES + 5]
                             * jnp.float32(stride), 0.0)


def _fused_kernel(*refs):
    x_refs = refs[0:NLEV]
    w3_ref, g2_ref, b3_ref, gam_ref, bet_ref, wf_ref, fb_ref = refs[NLEV:NLEV + 7]
    cls_refs = refs[NLEV + 7:NLEV + 7 + NLEV]
    reg_refs = refs[NLEV + 7 + NLEV:NLEV + 7 + 2 * NLEV]
    cent_refs = refs[NLEV + 7 + 2 * NLEV:]
    for l in range(NLEV):
        H, W = SIZES[l]
        _do_level(x_refs[l], w3_ref, g2_ref, b3_ref, gam_ref, bet_ref,
                  wf_ref, fb_ref, cls_refs[l], cent_refs[l], reg_refs[l],
                  H, W, STRIDES[l])


def kernel(fpn0, fpn1, fpn2, fpn3, fpn4,
           cls_w, cls_b, cls_g, cls_beta, cls_fw, cls_fb,
           reg_w, reg_b, reg_g, reg_beta, reg_fw, reg_fb):
    f32 = jnp.float32
    B = fpn0.shape[0]
    wtap = jnp.concatenate([cls_w, reg_w], axis=0)
    W3 = jnp.transpose(wtap, (0, 2, 3, 1)).reshape(HID, 9 * IN_CH)
    W3 = W3.astype(jnp.bfloat16)
    b3 = jnp.concatenate([cls_b, reg_b]).reshape(HID, 1).astype(f32)
    gam = jnp.concatenate([cls_g, reg_g]).reshape(HID, 1).astype(f32)
    bet = jnp.concatenate([cls_beta, reg_beta]).reshape(HID, 1).astype(f32)
    gidx = jnp.arange(2 * HID) // 3
    G2 = (gidx[:, None] == gidx[None, :]).astype(f32)
    Wf = jnp.zeros((OUT_ROWS, HID), f32)
    Wf = Wf.at[0:NUM_CLASSES, 0:IN_CH].set(cls_fw[:, :, 0, 0])
    Wf = Wf.at[NUM_CLASSES:NUM_CLASSES + 5, IN_CH:HID].set(reg_fw[:, :, 0, 0])
    Wf = Wf.astype(jnp.bfloat16)
    fb = jnp.zeros((OUT_ROWS, 1), f32)
    fb = fb.at[0:NUM_CLASSES, 0].set(cls_fb)
    fb = fb.at[NUM_CLASSES:NUM_CLASSES + 5, 0].set(reg_fb)

    xs = [x.reshape(B, IN_CH, h * w)
          for x, (h, w) in zip((fpn0, fpn1, fpn2, fpn3, fpn4), SIZES)]
    full = lambda shp: pl.BlockSpec(shp, lambda b: (0,) * len(shp))
    bspec = lambda c, s: pl.BlockSpec((1, c, s), lambda b: (b, 0, 0))
    out_shape = (
        tuple(jax.ShapeDtypeStruct((B, NUM_CLASSES, h * w), f32)
              for h, w in SIZES)
        + tuple(jax.ShapeDtypeStruct((B, 4, h * w), f32) for h, w in SIZES)
        + tuple(jax.ShapeDtypeStruct((B, 1, h * w), f32) for h, w in SIZES)
    )
    outs = pl.pallas_call(
        _fused_kernel,
        grid=(B,),
        in_specs=(
            [bspec(IN_CH, h * w) for h, w in SIZES]
            + [full((HID, 9 * IN_CH)), full((2 * HID, 2 * HID)),
               full((HID, 1)), full((HID, 1)), full((HID, 1)),
               full((OUT_ROWS, HID)), full((OUT_ROWS, 1))]
        ),
        out_specs=(
            tuple(bspec(NUM_CLASSES, h * w) for h, w in SIZES)
            + tuple(bspec(4, h * w) for h, w in SIZES)
            + tuple(bspec(1, h * w) for h, w in SIZES)
        ),
        out_shape=out_shape,
    )(*xs, W3, G2, b3, gam, bet, Wf, fb)
    cls_out = [o.reshape(B, NUM_CLASSES, h, w)
               for o, (h, w) in zip(outs[0:NLEV], SIZES)]
    reg_out = [o.reshape(B, 4, h, w)
               for o, (h, w) in zip(outs[NLEV:2 * NLEV], SIZES)]
    cent_out = [o.reshape(B, 1, h, w)
                for o, (h, w) in zip(outs[2 * NLEV:], SIZES)]
    return tuple(cls_out) + tuple(reg_out) + tuple(cent_out)
